# baseline (speedup 1.0000x reference)
import jax
import jax.numpy as jnp
from jax.experimental import pallas as pl
from jax.experimental.pallas import tpu as pltpu

CHUNKS = (1024, 1024, 1024, 512, 256, 256)
STARTS = tuple(sum(CHUNKS[:i]) for i in range(len(CHUNKS)))
NBUF = 3
MAXC = max(CHUNKS)


def _body(x_ref, b_ref, w_hbm, o_hbm, *scratch):
    n = len(CHUNKS)
    wbufs = scratch[0:NBUF]
    obufs = scratch[NBUF:NBUF + n]
    wsems = scratch[NBUF + n:NBUF + 2 * n]
    osems = scratch[NBUF + 2 * n:NBUF + 3 * n]

    def wcopy(i):
        s, c = STARTS[i], CHUNKS[i]
        return pltpu.make_async_copy(
            w_hbm.at[pl.ds(s, c), :], wbufs[i % NBUF].at[pl.ds(0, c), :],
            wsems[i])

    def ocopy(i):
        s, c = STARTS[i], CHUNKS[i]
        return pltpu.make_async_copy(
            obufs[i], o_hbm.at[:, pl.ds(s, c)], osems[i])

    for i in range(min(NBUF, n)):
        wcopy(i).start()
    for i in range(n):
        s, c = STARTS[i], CHUNKS[i]
        wcopy(i).wait()
        wv = wbufs[i % NBUF][pl.ds(0, c), :]
        acc = jax.lax.dot_general(
            x_ref[...], wv,
            dimension_numbers=(((1,), (1,)), ((), ())),
            preferred_element_type=jnp.float32,
            precision=jax.lax.Precision.DEFAULT,
        )
        obufs[i][...] = jnp.maximum(acc + b_ref[:, pl.ds(s, c)], 0.0)
        ocopy(i).start()
        if i + NBUF < n:
            wcopy(i + NBUF).start()
    for i in range(n):
        ocopy(i).wait()


def kernel(x, W, b):
    M, K = x.shape
    N = W.shape[0]
    b2 = b.reshape(1, N)
    scratch = (
        [pltpu.VMEM((MAXC, K), jnp.float32)] * NBUF
        + [pltpu.VMEM((M, c), jnp.float32) for c in CHUNKS]
        + [pltpu.SemaphoreType.DMA] * (2 * len(CHUNKS))
    )
    out = pl.pallas_call(
        _body,
        in_specs=[
            pl.BlockSpec((M, K), lambda: (0, 0)),
            pl.BlockSpec((1, N), lambda: (0, 0)),
            pl.BlockSpec(memory_space=pltpu.MemorySpace.HBM),
        ],
        out_specs=pl.BlockSpec(memory_space=pltpu.MemorySpace.HBM),
        out_shape=jax.ShapeDtypeStruct((M, N), jnp.float32),
        scratch_shapes=scratch,
    )(x, b2, W)
    return out
